# baseline (speedup 1.0000x reference)
import functools

import jax
import jax.numpy as jnp
from jax import lax
from jax.experimental import pallas as pl
from jax.experimental.pallas import tpu as pltpu
from jax.experimental.pallas import tpu_sc as plsc

N = 10000
E = 320000
D = 128

NC = 2
NS = 16
NW = NC * NS
EPT = E // NW
CH = 64
EBLK = 16
NBLK = 10
EPT_PAD = NBLK * EBLK * CH
ROWS_PER_TILE = 624
REM_ROWS = N - NS * ROWS_PER_TILE


def _spmm_body(x_hbm, src_hbm, dst_hbm, val_hbm, out_hbm,
               src_v, dst_v, val_v, rows0_v, rows1_v, rows2_v, acc_sh,
               g0, g1, g2, s0, s1, s2):
    c = lax.axis_index("c")
    s = lax.axis_index("s")
    gid = c * NS + s
    rows = (rows0_v, rows1_v, rows2_v)
    gsem = (g0, g1, g2)
    ssem = (s0, s1, s2)

    def _zero_row(i, carry):
        for j in range(D // 16):
            rows0_v[i, pl.ds(j * 16, 16)] = jnp.zeros((16,), jnp.float32)
        return carry
    lax.fori_loop(0, CH, _zero_row, 0)
    base = s * ROWS_PER_TILE
    for r in range(ROWS_PER_TILE // CH):
        pltpu.sync_copy(rows0_v, acc_sh.at[pl.ds(base + r * CH, CH)])
    rem = ROWS_PER_TILE % CH
    if rem:
        pltpu.sync_copy(rows0_v.at[pl.ds(0, rem)],
                        acc_sh.at[pl.ds(base + (ROWS_PER_TILE // CH) * CH, rem)])

    @pl.when(s == 0)
    def _zero_tail():
        pltpu.sync_copy(rows0_v.at[pl.ds(0, REM_ROWS)],
                        acc_sh.at[pl.ds(NS * ROWS_PER_TILE, REM_ROWS)])
    plsc.subcore_barrier()

    def _scale(buf, k):
        def _grp(g, c2):
            vv = val_v[k, pl.ds(g * 16, 16)]
            for l in range(16):
                v = jnp.full((16,), vv[l], jnp.float32)
                i = g * 16 + l
                for j in range(D // 16):
                    sl = (i, pl.ds(j * 16, 16))
                    buf[sl] = buf[sl] * v
            return c2
        lax.fori_loop(0, CH // 16, _grp, 0)

    def _gather_start(k, b):
        pltpu.async_copy(x_hbm.at[src_v.at[k]], rows[b], gsem[b])

    def _gather_wait(k, b):
        pltpu.make_async_copy(x_hbm.at[src_v.at[k]], rows[b],
                              gsem[b]).wait()

    def _scatter_start(k, b):
        pltpu.async_copy(rows[b], acc_sh.at[dst_v.at[k]], ssem[b],
                         add=True)

    def _scatter_wait(k, b):
        pltpu.make_async_copy(rows[b], acc_sh.at[dst_v.at[k]],
                              ssem[b]).wait()

    def _chunk_body(k, b, wait_scatter, issue_gather):
        nb = (b + 1) % 3
        _gather_wait(k, b)
        if wait_scatter:
            _scatter_wait(k - 2, nb)
        if issue_gather:
            _gather_start(k + 1, nb)
        _scale(rows[b], k)
        _scatter_start(k, b)

    def _block(blk, carry):
        pltpu.sync_copy(src_hbm.at[gid, blk], src_v)
        pltpu.sync_copy(dst_hbm.at[gid, blk], dst_v)
        pltpu.sync_copy(val_hbm.at[gid, blk], val_v)

        _gather_start(0, 0)
        _chunk_body(0, 0, wait_scatter=False, issue_gather=True)
        _chunk_body(1, 1, wait_scatter=False, issue_gather=True)

        def _step(st, c1):
            k = 3 * st + 2
            _chunk_body(k, 2, wait_scatter=True, issue_gather=True)
            _chunk_body(k + 1, 0, wait_scatter=True, issue_gather=True)
            _chunk_body(k + 2, 1, wait_scatter=True, issue_gather=True)
            return c1
        lax.fori_loop(0, (EBLK - 4) // 3, _step, 0)

        _chunk_body(EBLK - 2, 2, wait_scatter=True, issue_gather=True)
        _chunk_body(EBLK - 1, 0, wait_scatter=False, issue_gather=False)
        _scatter_wait(EBLK - 3, 1)
        _scatter_wait(EBLK - 2, 2)
        _scatter_wait(EBLK - 1, 0)
        return carry
    lax.fori_loop(0, NBLK, _block, 0)

    plsc.subcore_barrier()
    pltpu.sync_copy(acc_sh.at[pl.ds(base, ROWS_PER_TILE)],
                    out_hbm.at[c, pl.ds(base, ROWS_PER_TILE)])

    @pl.when(s == 0)
    def _write_tail():
        pltpu.sync_copy(acc_sh.at[pl.ds(NS * ROWS_PER_TILE, REM_ROWS)],
                        out_hbm.at[c, pl.ds(NS * ROWS_PER_TILE, REM_ROWS)])


@functools.partial(
    pl.kernel,
    out_type=jax.ShapeDtypeStruct((NC, N, D), jnp.float32),
    mesh=plsc.VectorSubcoreMesh(core_axis_name="c", subcore_axis_name="s"),
    scratch_types=[
        pltpu.VMEM((EBLK, CH), jnp.int32),
        pltpu.VMEM((EBLK, CH), jnp.int32),
        pltpu.VMEM((EBLK, CH), jnp.float32),
        pltpu.VMEM((CH, D), jnp.float32),
        pltpu.VMEM((CH, D), jnp.float32),
        pltpu.VMEM((CH, D), jnp.float32),
        pltpu.VMEM_SHARED((N, D), jnp.float32),
        pltpu.SemaphoreType.DMA,
        pltpu.SemaphoreType.DMA,
        pltpu.SemaphoreType.DMA,
        pltpu.SemaphoreType.DMA,
        pltpu.SemaphoreType.DMA,
        pltpu.SemaphoreType.DMA,
    ],
)
def _spmm(x_hbm, src_hbm, dst_hbm, val_hbm, out_hbm,
          src_v, dst_v, val_v, rows0_v, rows1_v, rows2_v, acc_sh,
          g0, g1, g2, s0, s1, s2):
    _spmm_body(x_hbm, src_hbm, dst_hbm, val_hbm, out_hbm,
               src_v, dst_v, val_v, rows0_v, rows1_v, rows2_v, acc_sh,
               g0, g1, g2, s0, s1, s2)


def _matmul_body(p_ref, w_ref, b_ref, o_ref):
    z = p_ref[0] + p_ref[1]
    o_ref[...] = jnp.dot(z, w_ref[...],
                         preferred_element_type=jnp.float32) + b_ref[...]


def _matmul(partials, W, bias):
    grid = 5
    bm = N // grid
    return pl.pallas_call(
        _matmul_body,
        grid=(grid,),
        in_specs=[
            pl.BlockSpec((NC, bm, D), lambda i: (0, i, 0)),
            pl.BlockSpec((D, D), lambda i: (0, 0)),
            pl.BlockSpec((1, D), lambda i: (0, 0)),
        ],
        out_specs=pl.BlockSpec((bm, D), lambda i: (i, 0)),
        out_shape=jax.ShapeDtypeStruct((N, D), jnp.float32),
    )(partials, W, bias.reshape(1, D))


def kernel(x, edge_index, edge_vals, W, bias):
    ei = edge_index.astype(jnp.int32)
    npad = EPT_PAD - EPT
    pad = ((0, 0), (0, npad))
    dpad = jnp.broadcast_to((jnp.arange(npad, dtype=jnp.int32) * 41) % N,
                            (NW, npad))
    src = jnp.pad(ei[0].reshape(NW, EPT), pad).reshape(NW, NBLK, EBLK, CH)
    dst = jnp.concatenate([ei[1].reshape(NW, EPT), dpad],
                          axis=1).reshape(NW, NBLK, EBLK, CH)
    val = jnp.pad(edge_vals.reshape(NW, EPT), pad).reshape(NW, NBLK, EBLK, CH)
    partials = _spmm(x, src, dst, val)
    return _matmul(partials, W, bias)


# speedup vs baseline: 2.4937x; 2.4937x over previous
import functools

import jax
import jax.numpy as jnp
from jax import lax
from jax.experimental import pallas as pl
from jax.experimental.pallas import tpu as pltpu
from jax.experimental.pallas import tpu_sc as plsc

N = 10000
E = 320000
D = 128

NC = 2
NS = 16
NW = NC * NS
EPT = E // NW
CH = 80
NCHUNK = EPT // CH
EBLK = 25
NBLK = NCHUNK // EBLK
ROWS_PER_TILE = 624
REM_ROWS = N - NS * ROWS_PER_TILE


def _spmm_body(x_hbm, src_hbm, dst_hbm, val_hbm, out_hbm,
               src_v, dst_v, val_v, rows0_v, rows1_v, rows2_v, acc_sh,
               g0, g1, g2, s0, s1, s2):
    c = lax.axis_index("c")
    s = lax.axis_index("s")
    gid = c * NS + s
    rows = (rows0_v, rows1_v, rows2_v)
    gsem = (g0, g1, g2)
    ssem = (s0, s1, s2)

    def _zero_row(i, carry):
        for j in range(D // 16):
            rows0_v[i, pl.ds(j * 16, 16)] = jnp.zeros((16,), jnp.float32)
        return carry
    lax.fori_loop(0, CH, _zero_row, 0)
    base = s * ROWS_PER_TILE
    for r in range(ROWS_PER_TILE // CH):
        pltpu.sync_copy(rows0_v, acc_sh.at[pl.ds(base + r * CH, CH)])
    rem = ROWS_PER_TILE % CH
    if rem:
        pltpu.sync_copy(rows0_v.at[pl.ds(0, rem)],
                        acc_sh.at[pl.ds(base + (ROWS_PER_TILE // CH) * CH, rem)])

    @pl.when(s == 0)
    def _zero_tail():
        pltpu.sync_copy(rows0_v.at[pl.ds(0, REM_ROWS)],
                        acc_sh.at[pl.ds(NS * ROWS_PER_TILE, REM_ROWS)])
    plsc.subcore_barrier()

    def _scale(buf, k):
        def _grp(g, c2):
            vv = val_v[k, pl.ds(g * 16, 16)]
            for l in range(16):
                v = jnp.full((16,), vv[l], jnp.float32)
                i = g * 16 + l
                for j in range(D // 16):
                    sl = (i, pl.ds(j * 16, 16))
                    buf[sl] = buf[sl] * v
            return c2
        lax.fori_loop(0, CH // 16, _grp, 0)

    def _gather_start(k, b):
        pltpu.async_copy(x_hbm.at[src_v.at[k]], rows[b], gsem[b])

    def _gather_wait(k, b):
        pltpu.make_async_copy(x_hbm.at[src_v.at[k]], rows[b],
                              gsem[b]).wait()

    def _scatter_start(k, b):
        pltpu.async_copy(rows[b], acc_sh.at[dst_v.at[k]], ssem[b],
                         add=True)

    def _scatter_wait(k, b):
        pltpu.make_async_copy(rows[b], acc_sh.at[dst_v.at[k]],
                              ssem[b]).wait()

    def _chunk_body(k, b, wait_scatter, issue_gather):
        nb = (b + 1) % 3
        _gather_wait(k, b)
        if wait_scatter:
            _scatter_wait(k - 2, nb)
        if issue_gather:
            _gather_start(k + 1, nb)
        _scale(rows[b], k)
        _scatter_start(k, b)

    def _block(blk, carry):
        pltpu.sync_copy(src_hbm.at[gid, blk], src_v)
        pltpu.sync_copy(dst_hbm.at[gid, blk], dst_v)
        pltpu.sync_copy(val_hbm.at[gid, blk], val_v)

        _gather_start(0, 0)
        _chunk_body(0, 0, wait_scatter=False, issue_gather=True)
        _chunk_body(1, 1, wait_scatter=False, issue_gather=True)

        def _step(st, c1):
            k = 3 * st + 2
            _chunk_body(k, 2, wait_scatter=True, issue_gather=True)
            _chunk_body(k + 1, 0, wait_scatter=True, issue_gather=True)
            _chunk_body(k + 2, 1, wait_scatter=True, issue_gather=True)
            return c1
        lax.fori_loop(0, (EBLK - 4) // 3, _step, 0)

        _chunk_body(EBLK - 2, 2, wait_scatter=True, issue_gather=True)
        _chunk_body(EBLK - 1, 0, wait_scatter=False, issue_gather=False)
        _scatter_wait(EBLK - 3, 1)
        _scatter_wait(EBLK - 2, 2)
        _scatter_wait(EBLK - 1, 0)
        return carry
    lax.fori_loop(0, NBLK, _block, 0)

    plsc.subcore_barrier()
    pltpu.sync_copy(acc_sh.at[pl.ds(base, ROWS_PER_TILE)],
                    out_hbm.at[c, pl.ds(base, ROWS_PER_TILE)])

    @pl.when(s == 0)
    def _write_tail():
        pltpu.sync_copy(acc_sh.at[pl.ds(NS * ROWS_PER_TILE, REM_ROWS)],
                        out_hbm.at[c, pl.ds(NS * ROWS_PER_TILE, REM_ROWS)])


@functools.partial(
    pl.kernel,
    out_type=jax.ShapeDtypeStruct((NC, N, D), jnp.float32),
    mesh=plsc.VectorSubcoreMesh(core_axis_name="c", subcore_axis_name="s"),
    scratch_types=[
        pltpu.VMEM((EBLK, CH), jnp.int32),
        pltpu.VMEM((EBLK, CH), jnp.int32),
        pltpu.VMEM((EBLK, CH), jnp.float32),
        pltpu.VMEM((CH, D), jnp.float32),
        pltpu.VMEM((CH, D), jnp.float32),
        pltpu.VMEM((CH, D), jnp.float32),
        pltpu.VMEM_SHARED((N, D), jnp.float32),
        pltpu.SemaphoreType.DMA,
        pltpu.SemaphoreType.DMA,
        pltpu.SemaphoreType.DMA,
        pltpu.SemaphoreType.DMA,
        pltpu.SemaphoreType.DMA,
        pltpu.SemaphoreType.DMA,
    ],
)
def _spmm(x_hbm, src_hbm, dst_hbm, val_hbm, out_hbm,
          src_v, dst_v, val_v, rows0_v, rows1_v, rows2_v, acc_sh,
          g0, g1, g2, s0, s1, s2):
    _spmm_body(x_hbm, src_hbm, dst_hbm, val_hbm, out_hbm,
               src_v, dst_v, val_v, rows0_v, rows1_v, rows2_v, acc_sh,
               g0, g1, g2, s0, s1, s2)


def _matmul_body(p_ref, w_ref, b_ref, o_ref):
    z = p_ref[0] + p_ref[1]
    o_ref[...] = jnp.dot(z, w_ref[...],
                         preferred_element_type=jnp.float32) + b_ref[...]


def _matmul(partials, W, bias):
    grid = 5
    bm = N // grid
    return pl.pallas_call(
        _matmul_body,
        grid=(grid,),
        in_specs=[
            pl.BlockSpec((NC, bm, D), lambda i: (0, i, 0)),
            pl.BlockSpec((D, D), lambda i: (0, 0)),
            pl.BlockSpec((1, D), lambda i: (0, 0)),
        ],
        out_specs=pl.BlockSpec((bm, D), lambda i: (i, 0)),
        out_shape=jax.ShapeDtypeStruct((N, D), jnp.float32),
    )(partials, W, bias.reshape(1, D))


def kernel(x, edge_index, edge_vals, W, bias):
    ei = edge_index.astype(jnp.int32)
    src = ei[0].reshape(NW, NBLK, EBLK, CH)
    dst = ei[1].reshape(NW, NBLK, EBLK, CH)
    val = edge_vals.reshape(NW, NBLK, EBLK, CH)
    partials = _spmm(x, src, dst, val)
    return _matmul(partials, W, bias)
